# baseline (speedup 1.0000x reference)
import functools

import jax
import jax.numpy as jnp
from jax import lax
from jax.experimental import pallas as pl
from jax.experimental.pallas import tpu as pltpu
from jax.experimental.pallas import tpu_sc as plsc

B, L, D = 16, 4096, 512
LANES = 16
NCORES, NSUB = 2, 16
NW = NCORES * NSUB
HALF = L // 2
CHUNK = 32
NCH = HALF // CHUNK
NB = 4
VPP = D // LANES


def _body(x_hbm, values_hbm, cand_hbm, buf, res_v, rsems, wsems, semw):
    c = lax.axis_index("c")
    s = lax.axis_index("s")
    w = s * NCORES + c
    b = w // 2
    h = w % 2
    base_pos = h * HALF

    zeros = jnp.zeros((LANES,), jnp.float32)
    lane = lax.iota(jnp.int32, LANES)

    def vmax_scalar(v):
        for k_ in (1, 2, 4, 8):
            v = jnp.maximum(v, jnp.take(v, lane ^ k_))
        return v[0]

    def rd(k, q):
        return pltpu.make_async_copy(
            x_hbm.at[b, pl.ds(base_pos + k * CHUNK, CHUNK), :],
            buf.at[pl.ds(q * CHUNK, CHUNK), :], rsems[q])

    def wr(k, q):
        return pltpu.make_async_copy(
            buf.at[pl.ds(q * CHUNK, CHUNK), :],
            values_hbm.at[b, pl.ds(base_pos + k * CHUNK, CHUNK), :], wsems[q])

    def absmax(q):
        def g_body(p, accs):
            accs = list(accs)
            for i in range(VPP):
                accs[i % 4] = jnp.maximum(
                    accs[i % 4],
                    jnp.abs(buf[q * CHUNK + p, pl.ds(i * LANES, LANES)]))
            return tuple(accs)
        a0, a1, a2, a3 = lax.fori_loop(
            0, CHUNK, g_body, (zeros, zeros, zeros, zeros))
        return jnp.maximum(jnp.maximum(a0, a1), jnp.maximum(a2, a3))

    def step(i, p, best, first):
        k = NCH - 1 - i
        if not first:
            wr(0, (p - 1) % NB).wait()
        kr = jnp.maximum(k - (NB - 1), 0)
        rd(kr, (p + NB - 1) % NB).start()
        rd(k, p).wait()

        def check(_):
            found = vmax_scalar(absmax(p)) > 0.0
            return jnp.where(found, k, best)

        best = lax.cond(best < 0, check, lambda _: best, 0)
        wr(k, p).start()
        return best

    for q in range(NB - 1):
        rd(NCH - 1 - q, q).start()

    best = jnp.int32(-1)
    for p in range(NB):
        best = step(jnp.int32(p), p, best, first=(p == 0))

    def group(g, best):
        for p in range(NB):
            best = step(g * NB + p, p, best, first=False)
        return best

    best = lax.fori_loop(1, NCH // NB, group, best)

    wr(0, (NCH - 1) % NB).wait()
    for q in range(NB - 1):
        rd(0, q).wait()

    def resolve(best_):
        rd(best_, 0).start()
        rd(best_, 0).wait()

        def p_body(p, last_p):
            acc = zeros
            for i in range(VPP):
                acc = jnp.maximum(
                    acc, jnp.abs(buf[p, pl.ds(i * LANES, LANES)]))
            nz = vmax_scalar(acc) > 0.0
            return jnp.where(nz, p, last_p)

        last_p = lax.fori_loop(0, CHUNK, p_body, jnp.int32(0))
        return base_pos + best_ * CHUNK + last_p + 1

    length = lax.cond(best >= 0, resolve, lambda _: jnp.int32(0), best)

    res_v[:] = jnp.where(lane == b, length, 0)
    cp = pltpu.make_async_copy(res_v, cand_hbm.at[w], semw)
    cp.start()
    cp.wait()


_fused_kernel = functools.partial(
    pl.kernel,
    out_type=(
        jax.ShapeDtypeStruct((B, L, D), jnp.float32),
        jax.ShapeDtypeStruct((NW, NSUB), jnp.int32),
    ),
    mesh=plsc.VectorSubcoreMesh(core_axis_name="c", subcore_axis_name="s"),
    scratch_types=[
        pltpu.VMEM((NB * CHUNK, D), jnp.float32),
        pltpu.VMEM((LANES,), jnp.int32),
        [pltpu.SemaphoreType.DMA] * NB,
        [pltpu.SemaphoreType.DMA] * NB,
        pltpu.SemaphoreType.DMA,
    ],
)(_body)


def kernel(inputs):
    values, cand = _fused_kernel(inputs.reshape(B, L, D))
    row_lengths = jnp.max(cand, axis=0).astype(jnp.int32)
    return (values, row_lengths)
